# baseline (speedup 1.0000x reference)
import functools

import jax
import jax.numpy as jnp
from jax import lax
from jax.experimental import pallas as pl
from jax.experimental.pallas import tpu as pltpu
from jax.experimental.pallas import tpu_sc as plsc

NN = 10000
NE = 320000
FIN = 128
F = 64
L = 16
NC, NS = 2, 16
NW = NC * NS
NPAD = 10240
RPT = NPAD // NS
EPW = NE // NW
CH = 128
NCH = 80
EPWP = NCH * CH
NEP = NW * EPWP
NROWS = 10016

_f32 = jnp.float32
_mesh = plsc.VectorSubcoreMesh(core_axis_name="c", subcore_axis_name="s")
_sc_params = pltpu.CompilerParams(use_tc_tiling_on_sc=False)


def _seg_body(with_deg, src_hbm, dst_hbm, pre_hbm, z64_hbm, *rest):
    if with_deg:
        (z16_hbm, ones_hbm, agg_out, deg_out,
         idxs, idxd, r0b, r1b, r2b, ones_v, acc_sh, deg_sh,
         g0, g1, g2, s0, s1, s2, d0, d1, d2) = rest
        dsem = (d0, d1, d2)
    else:
        (agg_out, idxs, idxd, r0b, r1b, r2b, acc_sh,
         g0, g1, g2, s0, s1, s2) = rest
    rows = (r0b, r1b, r2b)
    gs = (g0, g1, g2)
    ss = (s0, s1, s2)
    c = lax.axis_index("c")
    s = lax.axis_index("s")
    wid = s * NC + c
    r0 = s * RPT
    pltpu.sync_copy(z64_hbm.at[pl.ds(r0, RPT)], acc_sh.at[pl.ds(r0, RPT)])
    if with_deg:
        pltpu.sync_copy(z16_hbm.at[pl.ds(r0, RPT)], deg_sh.at[pl.ds(r0, RPT)])
        pltpu.sync_copy(ones_hbm, ones_v)
    pltpu.sync_copy(src_hbm.at[wid], idxs)
    pltpu.sync_copy(dst_hbm.at[wid], idxd)
    plsc.subcore_barrier()

    def g_start(j, b):
        pltpu.async_copy(pre_hbm.at[idxs.at[j]], rows[b], gs[b])

    def g_wait(j, b):
        pltpu.make_async_copy(pre_hbm.at[idxs.at[j]], rows[b], gs[b]).wait()

    def d_wait(j, b):
        pltpu.make_async_copy(ones_v, deg_sh.at[idxd.at[j]], dsem[b]).wait()

    def s_start(j, b):
        pltpu.async_copy(rows[b], acc_sh.at[idxd.at[j]], ss[b], add=True)
        if with_deg:
            pltpu.async_copy(ones_v, deg_sh.at[idxd.at[j]], dsem[b],
                             add=True)

    def s_wait(j, b):
        pltpu.make_async_copy(rows[b], acc_sh.at[idxd.at[j]], ss[b]).wait()

    g_start(0, 0)
    g_start(1, 1)

    @pl.loop(0, NCH - 2, step=3)
    def _loop(j2):
        for b in range(3):
            j = j2 + b
            g_wait(j, b)
            if with_deg:
                @pl.when(j2 >= 3)
                def _():
                    d_wait(j - 3, b)
            s_start(j, b)
            nb = (b + 2) % 3
            if b == 0:
                @pl.when(j2 >= 1)
                def _():
                    s_wait(j - 1, nb)
            else:
                s_wait(j - 1, nb)
            g_start(j + 2, nb)

    for j in (NCH - 2, NCH - 1):
        b = j % 3
        g_wait(j, b)
        if with_deg:
            d_wait(j - 3, b)
        s_start(j, b)
    for j in (NCH - 3, NCH - 2, NCH - 1):
        s_wait(j, j % 3)
        if with_deg:
            d_wait(j, j % 3)
    plsc.subcore_barrier()
    pltpu.sync_copy(acc_sh.at[pl.ds(r0, RPT)], agg_out.at[c].at[pl.ds(r0, RPT)])
    if with_deg:
        pltpu.sync_copy(deg_sh.at[pl.ds(r0, RPT)], deg_out.at[c].at[pl.ds(r0, RPT)])


_seg_deg = pl.kernel(
    functools.partial(_seg_body, True),
    out_type=(jax.ShapeDtypeStruct((NC, NPAD, F), _f32),
              jax.ShapeDtypeStruct((NC, NPAD, L), _f32)),
    mesh=_mesh,
    compiler_params=_sc_params,
    scratch_types=[
        pltpu.VMEM((NCH, CH), jnp.int32),
        pltpu.VMEM((NCH, CH), jnp.int32),
        pltpu.VMEM((CH, F), _f32),
        pltpu.VMEM((CH, F), _f32),
        pltpu.VMEM((CH, F), _f32),
        pltpu.VMEM((CH, L), _f32),
        pltpu.VMEM_SHARED((NPAD, F), _f32),
        pltpu.VMEM_SHARED((NPAD, L), _f32),
        pltpu.SemaphoreType.DMA,
        pltpu.SemaphoreType.DMA,
        pltpu.SemaphoreType.DMA,
        pltpu.SemaphoreType.DMA,
        pltpu.SemaphoreType.DMA,
        pltpu.SemaphoreType.DMA,
        pltpu.SemaphoreType.DMA,
        pltpu.SemaphoreType.DMA,
        pltpu.SemaphoreType.DMA,
    ],
)

_seg = pl.kernel(
    functools.partial(_seg_body, False),
    out_type=jax.ShapeDtypeStruct((NC, NPAD, F), _f32),
    mesh=_mesh,
    compiler_params=_sc_params,
    scratch_types=[
        pltpu.VMEM((NCH, CH), jnp.int32),
        pltpu.VMEM((NCH, CH), jnp.int32),
        pltpu.VMEM((CH, F), _f32),
        pltpu.VMEM((CH, F), _f32),
        pltpu.VMEM((CH, F), _f32),
        pltpu.VMEM_SHARED((NPAD, F), _f32),
        pltpu.SemaphoreType.DMA,
        pltpu.SemaphoreType.DMA,
        pltpu.SemaphoreType.DMA,
        pltpu.SemaphoreType.DMA,
        pltpu.SemaphoreType.DMA,
        pltpu.SemaphoreType.DMA,
    ],
)


def _score_body(srcs_hbm, dsts_hbm, a_hbm, b_hbm, w2_hbm, out_hbm,
                idxs, idxd, ra0, ra1, rb0, rb1, p0, p1, w2_v,
                ga0, ga1, gb0, gb1, os0, os1):
    rows_a = (ra0, ra1)
    rows_b = (rb0, rb1)
    part = (p0, p1)
    gsa = (ga0, ga1)
    gsb = (gb0, gb1)
    osem = (os0, os1)
    c = lax.axis_index("c")
    s = lax.axis_index("s")
    wid = s * NC + c
    base = wid * EPWP
    pltpu.sync_copy(srcs_hbm.at[wid], idxs)
    pltpu.sync_copy(dsts_hbm.at[wid], idxd)
    pltpu.sync_copy(w2_hbm, w2_v)
    w2s = tuple(w2_v[pl.ds(k * L, L)] for k in range(F // L))

    def g_start(j, b):
        pltpu.async_copy(a_hbm.at[idxs.at[j]], rows_a[b], gsa[b])
        pltpu.async_copy(b_hbm.at[idxd.at[j]], rows_b[b], gsb[b])

    def g_wait(j, b):
        pltpu.make_async_copy(a_hbm.at[idxs.at[j]], rows_a[b], gsa[b]).wait()
        pltpu.make_async_copy(b_hbm.at[idxd.at[j]], rows_b[b], gsb[b]).wait()

    def compute(b):
        U = 8

        def edges(i, carry):
            for u in range(U):
                e = i * U + u
                acc = jnp.zeros((L,), _f32)
                for k in range(F // L):
                    g = jnp.maximum(
                        rows_a[b][e, pl.ds(k * L, L)]
                        + rows_b[b][e, pl.ds(k * L, L)], 0.0)
                    acc = acc + g * w2s[k]
                part[b][e, :] = acc
            return carry

        lax.fori_loop(0, CH // U, edges, 0)

    def o_start(j, b):
        pltpu.async_copy(part[b], out_hbm.at[pl.ds(base + j * CH, CH)],
                         osem[b])

    def o_wait(j, b):
        pltpu.make_async_copy(part[b], out_hbm.at[pl.ds(base + j * CH, CH)],
                              osem[b]).wait()

    g_start(0, 0)
    g_start(1, 1)

    @pl.loop(0, NCH - 2, step=2)
    def _loop(j2):
        for b in range(2):
            j = j2 + b

            @pl.when(j2 >= 2)
            def _():
                o_wait(j - 2, b)

            g_wait(j, b)
            compute(b)
            o_start(j, b)
            g_start(j + 2, b)

    for j in (NCH - 2, NCH - 1):
        b = j % 2
        o_wait(j - 2, b)
        g_wait(j, b)
        compute(b)
        o_start(j, b)
    o_wait(NCH - 2, 0)
    o_wait(NCH - 1, 1)


_score = pl.kernel(
    _score_body,
    out_type=jax.ShapeDtypeStruct((NEP, L), _f32),
    mesh=_mesh,
    compiler_params=_sc_params,
    scratch_types=[
        pltpu.VMEM((NCH, CH), jnp.int32),
        pltpu.VMEM((NCH, CH), jnp.int32),
        pltpu.VMEM((CH, F), _f32),
        pltpu.VMEM((CH, F), _f32),
        pltpu.VMEM((CH, F), _f32),
        pltpu.VMEM((CH, F), _f32),
        pltpu.VMEM((CH, L), _f32),
        pltpu.VMEM((CH, L), _f32),
        pltpu.VMEM((F,), _f32),
        pltpu.SemaphoreType.DMA,
        pltpu.SemaphoreType.DMA,
        pltpu.SemaphoreType.DMA,
        pltpu.SemaphoreType.DMA,
        pltpu.SemaphoreType.DMA,
        pltpu.SemaphoreType.DMA,
    ],
)


def _k1_body(x_ref, wn_ref, ws_ref, bs_ref, pre_ref, self_ref):
    xv = x_ref[...]
    pre_ref[:NN, :] = jnp.dot(xv, wn_ref[...], preferred_element_type=_f32)
    pre_ref[NN:, :] = jnp.zeros((NROWS - NN, F), _f32)
    self_ref[...] = (jnp.dot(xv, ws_ref[...], preferred_element_type=_f32)
                     + bs_ref[...])


_k1 = pl.pallas_call(
    _k1_body,
    out_shape=(jax.ShapeDtypeStruct((NROWS, F), _f32),
               jax.ShapeDtypeStruct((NN, F), _f32)),
)


def _mix_body(self_ref, agg_ref, deg_ref, wn_ref, ws_ref, bs_ref,
              pre_ref, self2_ref):
    deg = deg_ref[0, :NN, 0:1] + deg_ref[1, :NN, 0:1]
    agg = agg_ref[0, :NN, :] + agg_ref[1, :NN, :]
    h = jnp.maximum(self_ref[...] + agg / jnp.maximum(deg, 1.0), 0.0)
    pre_ref[:NN, :] = jnp.dot(h, wn_ref[...], preferred_element_type=_f32)
    pre_ref[NN:, :] = jnp.zeros((NROWS - NN, F), _f32)
    self2_ref[...] = (jnp.dot(h, ws_ref[...], preferred_element_type=_f32)
                      + bs_ref[...])


_k2 = pl.pallas_call(
    _mix_body,
    out_shape=(jax.ShapeDtypeStruct((NROWS, F), _f32),
               jax.ShapeDtypeStruct((NN, F), _f32)),
)


def _k3_body(self_ref, agg_ref, deg_ref, wa_ref, wb_ref, bm_ref,
             h_ref, a_ref, b_ref):
    deg = deg_ref[0, :NN, 0:1] + deg_ref[1, :NN, 0:1]
    agg = agg_ref[0, :NN, :] + agg_ref[1, :NN, :]
    h = jnp.maximum(self_ref[...] + agg / jnp.maximum(deg, 1.0), 0.0)
    h_ref[...] = h
    a_ref[:NN, :] = (jnp.dot(h, wa_ref[...], preferred_element_type=_f32)
                     + bm_ref[...])
    a_ref[NN:, :] = jnp.zeros((NROWS - NN, F), _f32)
    b_ref[:NN, :] = jnp.dot(h, wb_ref[...], preferred_element_type=_f32)
    b_ref[NN:, :] = jnp.zeros((NROWS - NN, F), _f32)


_k3 = pl.pallas_call(
    _k3_body,
    out_shape=(jax.ShapeDtypeStruct((NN, F), _f32),
               jax.ShapeDtypeStruct((NROWS, F), _f32),
               jax.ShapeDtypeStruct((NROWS, F), _f32)),
)


def _fin_body(p_ref, b2_ref, out_ref):
    i = lax.broadcasted_iota(jnp.int32, (128, 8), 0) // L
    j = lax.broadcasted_iota(jnp.int32, (128, 8), 1)
    sel = (i == j).astype(_f32)
    z = jnp.dot(p_ref[...], sel, preferred_element_type=_f32) + b2_ref[...]
    out_ref[...] = jax.nn.sigmoid(z)


_FINR = NEP * L // 128
_fin = pl.pallas_call(
    _fin_body,
    grid=(5,),
    in_specs=[pl.BlockSpec((_FINR // 5, 128), lambda i: (i, 0)),
              pl.BlockSpec((1, 1), lambda i: (0, 0))],
    out_specs=pl.BlockSpec((_FINR // 5, 8), lambda i: (i, 0)),
    out_shape=jax.ShapeDtypeStruct((_FINR, 8), _f32),
)


def _pad_idx(v):
    v = v.astype(jnp.int32).reshape(NW, EPW)
    v = jnp.pad(v, ((0, 0), (0, EPWP - EPW)), constant_values=NN)
    return v.reshape(NW, NCH, CH)


def kernel(x, W_self1, b_self1, W_neigh1, W_self2, b_self2, W_neigh2,
           W_mlp1, b_mlp1, W_mlp2, b_mlp2, edge_index, score_edge_index):
    src = _pad_idx(edge_index[0])
    dst = _pad_idx(edge_index[1])
    ssrc = _pad_idx(score_edge_index[0])
    sdst = _pad_idx(score_edge_index[1])
    z64 = jnp.zeros((NPAD, F), _f32)
    z16 = jnp.zeros((NPAD, L), _f32)
    ones = jnp.ones((CH, L), _f32)

    pre1, self1 = _k1(x, W_neigh1, W_self1, b_self1.reshape(1, F))
    agg1, deg = _seg_deg(src, dst, pre1, z64, z16, ones)
    pre2, self2 = _k2(self1, agg1, deg, W_neigh2, W_self2,
                      b_self2.reshape(1, F))
    agg2 = _seg(src, dst, pre2, z64)
    h, A, B = _k3(self2, agg2, deg, W_mlp1[:F], W_mlp1[F:],
                  b_mlp1.reshape(1, F))
    parts = _score(ssrc, sdst, A, B, W_mlp2.reshape(F))
    score = _fin(parts.reshape(_FINR, 128), b_mlp2.reshape(1, 1))
    score = score.reshape(NW, EPWP)[:, :EPW].reshape(NE, 1)
    return (score, h)


# speedup vs baseline: 2.1493x; 2.1493x over previous
import functools

import jax
import jax.numpy as jnp
from jax import lax
from jax.experimental import pallas as pl
from jax.experimental.pallas import tpu as pltpu
from jax.experimental.pallas import tpu_sc as plsc

NN = 10000
NE = 320000
FIN = 128
F = 64
L = 16
NC, NS = 2, 16
NW = NC * NS
NPAD = 10240
RPT = NPAD // NS
EPW = NE // NW
CH = 80
NCH = EPW // CH

_f32 = jnp.float32
_mesh = plsc.VectorSubcoreMesh(core_axis_name="c", subcore_axis_name="s")
_sc_params = pltpu.CompilerParams(use_tc_tiling_on_sc=False)


def _seg_body(with_deg, src_hbm, dst_hbm, pre_hbm, z64_hbm, *rest):
    if with_deg:
        (z16_hbm, ones_hbm, agg_out, deg_out,
         idxs, idxd, r0b, r1b, r2b, ones_v, acc_sh, deg_sh,
         g0, g1, g2, s0, s1, s2, d0, d1, d2) = rest
        dsem = (d0, d1, d2)
    else:
        (agg_out, idxs, idxd, r0b, r1b, r2b, acc_sh,
         g0, g1, g2, s0, s1, s2) = rest
    rows = (r0b, r1b, r2b)
    gs = (g0, g1, g2)
    ss = (s0, s1, s2)
    c = lax.axis_index("c")
    s = lax.axis_index("s")
    wid = s * NC + c
    r0 = s * RPT
    pltpu.sync_copy(z64_hbm.at[pl.ds(r0, RPT)], acc_sh.at[pl.ds(r0, RPT)])
    if with_deg:
        pltpu.sync_copy(z16_hbm.at[pl.ds(r0, RPT)], deg_sh.at[pl.ds(r0, RPT)])
        pltpu.sync_copy(ones_hbm, ones_v)
    pltpu.sync_copy(src_hbm.at[wid], idxs)
    pltpu.sync_copy(dst_hbm.at[wid], idxd)
    plsc.subcore_barrier()

    def g_start(j, b):
        pltpu.async_copy(pre_hbm.at[idxs.at[j]], rows[b], gs[b])

    def g_wait(j, b):
        pltpu.make_async_copy(pre_hbm.at[idxs.at[j]], rows[b], gs[b]).wait()

    def d_wait(j, b):
        pltpu.make_async_copy(ones_v, deg_sh.at[idxd.at[j]], dsem[b]).wait()

    def s_start(j, b):
        pltpu.async_copy(rows[b], acc_sh.at[idxd.at[j]], ss[b], add=True)
        if with_deg:
            pltpu.async_copy(ones_v, deg_sh.at[idxd.at[j]], dsem[b],
                             add=True)

    def s_wait(j, b):
        pltpu.make_async_copy(rows[b], acc_sh.at[idxd.at[j]], ss[b]).wait()

    g_start(0, 0)
    g_start(1, 1)

    @pl.loop(0, NCH - 2, step=3)
    def _loop(j2):
        for b in range(3):
            j = j2 + b
            g_wait(j, b)
            if with_deg:
                @pl.when(j2 >= 3)
                def _():
                    d_wait(j - 3, b)
            s_start(j, b)
            nb = (b + 2) % 3
            if b == 0:
                @pl.when(j2 >= 1)
                def _():
                    s_wait(j - 1, nb)
            else:
                s_wait(j - 1, nb)
            g_start(j + 2, nb)

    for j in (NCH - 2, NCH - 1):
        b = j % 3
        g_wait(j, b)
        if with_deg:
            d_wait(j - 3, b)
        s_start(j, b)
    for j in (NCH - 3, NCH - 2, NCH - 1):
        s_wait(j, j % 3)
        if with_deg:
            d_wait(j, j % 3)
    plsc.subcore_barrier()
    pltpu.sync_copy(acc_sh.at[pl.ds(r0, RPT)], agg_out.at[c].at[pl.ds(r0, RPT)])
    if with_deg:
        pltpu.sync_copy(deg_sh.at[pl.ds(r0, RPT)], deg_out.at[c].at[pl.ds(r0, RPT)])


_seg_deg = pl.kernel(
    functools.partial(_seg_body, True),
    out_type=(jax.ShapeDtypeStruct((NC, NPAD, F), _f32),
              jax.ShapeDtypeStruct((NC, NPAD, L), _f32)),
    mesh=_mesh,
    compiler_params=_sc_params,
    scratch_types=[
        pltpu.VMEM((NCH, CH), jnp.int32),
        pltpu.VMEM((NCH, CH), jnp.int32),
        pltpu.VMEM((CH, F), _f32),
        pltpu.VMEM((CH, F), _f32),
        pltpu.VMEM((CH, F), _f32),
        pltpu.VMEM((CH, L), _f32),
        pltpu.VMEM_SHARED((NPAD, F), _f32),
        pltpu.VMEM_SHARED((NPAD, L), _f32),
        pltpu.SemaphoreType.DMA,
        pltpu.SemaphoreType.DMA,
        pltpu.SemaphoreType.DMA,
        pltpu.SemaphoreType.DMA,
        pltpu.SemaphoreType.DMA,
        pltpu.SemaphoreType.DMA,
        pltpu.SemaphoreType.DMA,
        pltpu.SemaphoreType.DMA,
        pltpu.SemaphoreType.DMA,
    ],
)

_seg = pl.kernel(
    functools.partial(_seg_body, False),
    out_type=jax.ShapeDtypeStruct((NC, NPAD, F), _f32),
    mesh=_mesh,
    compiler_params=_sc_params,
    scratch_types=[
        pltpu.VMEM((NCH, CH), jnp.int32),
        pltpu.VMEM((NCH, CH), jnp.int32),
        pltpu.VMEM((CH, F), _f32),
        pltpu.VMEM((CH, F), _f32),
        pltpu.VMEM((CH, F), _f32),
        pltpu.VMEM_SHARED((NPAD, F), _f32),
        pltpu.SemaphoreType.DMA,
        pltpu.SemaphoreType.DMA,
        pltpu.SemaphoreType.DMA,
        pltpu.SemaphoreType.DMA,
        pltpu.SemaphoreType.DMA,
        pltpu.SemaphoreType.DMA,
    ],
)


def _score_body(srcs_hbm, dsts_hbm, a_hbm, b_hbm, w2_hbm, out_hbm,
                idxs, idxd, ra0, ra1, rb0, rb1, p0, p1, w2_v,
                ga0, ga1, gb0, gb1, os0, os1):
    rows_a = (ra0, ra1)
    rows_b = (rb0, rb1)
    part = (p0, p1)
    gsa = (ga0, ga1)
    gsb = (gb0, gb1)
    osem = (os0, os1)
    c = lax.axis_index("c")
    s = lax.axis_index("s")
    wid = s * NC + c
    base = wid * EPW
    pltpu.sync_copy(srcs_hbm.at[wid], idxs)
    pltpu.sync_copy(dsts_hbm.at[wid], idxd)
    pltpu.sync_copy(w2_hbm, w2_v)
    w2s = tuple(w2_v[pl.ds(k * L, L)] for k in range(F // L))

    def g_start(j, b):
        pltpu.async_copy(a_hbm.at[idxs.at[j]], rows_a[b], gsa[b])
        pltpu.async_copy(b_hbm.at[idxd.at[j]], rows_b[b], gsb[b])

    def g_wait(j, b):
        pltpu.make_async_copy(a_hbm.at[idxs.at[j]], rows_a[b], gsa[b]).wait()
        pltpu.make_async_copy(b_hbm.at[idxd.at[j]], rows_b[b], gsb[b]).wait()

    def compute(b):
        U = 8

        def edges(i, carry):
            accs = []
            for u in range(U):
                e = i * U + u
                acc = jnp.zeros((L,), _f32)
                for k in range(F // L):
                    g = jnp.maximum(
                        rows_a[b][e, pl.ds(k * L, L)]
                        + rows_b[b][e, pl.ds(k * L, L)], 0.0)
                    acc = acc + g * w2s[k]
                accs.append(acc)
            for u in range(U):
                part[b][i * U + u, :] = accs[u]
            return carry

        lax.fori_loop(0, CH // U, edges, 0)

    def o_start(j, b):
        pltpu.async_copy(part[b], out_hbm.at[pl.ds(base + j * CH, CH)],
                         osem[b])

    def o_wait(j, b):
        pltpu.make_async_copy(part[b], out_hbm.at[pl.ds(base + j * CH, CH)],
                              osem[b]).wait()

    g_start(0, 0)
    g_start(1, 1)

    @pl.loop(0, NCH - 1, step=2)
    def _loop(j2):
        for b in range(2):
            j = j2 + b

            @pl.when(j2 >= 2)
            def _():
                o_wait(j - 2, b)

            g_wait(j, b)
            compute(b)
            o_start(j, b)
            if b == 0:
                g_start(j + 2, b)
            else:
                @pl.when(j2 + 3 < NCH)
                def _():
                    g_start(j + 2, b)

    j = NCH - 1
    o_wait(j - 2, 0)
    g_wait(j, 0)
    compute(0)
    o_start(j, 0)
    o_wait(j - 1, 1)
    o_wait(j, 0)


_score = pl.kernel(
    _score_body,
    out_type=jax.ShapeDtypeStruct((NE, L), _f32),
    mesh=_mesh,
    compiler_params=_sc_params,
    scratch_types=[
        pltpu.VMEM((NCH, CH), jnp.int32),
        pltpu.VMEM((NCH, CH), jnp.int32),
        pltpu.VMEM((CH, F), _f32),
        pltpu.VMEM((CH, F), _f32),
        pltpu.VMEM((CH, F), _f32),
        pltpu.VMEM((CH, F), _f32),
        pltpu.VMEM((CH, L), _f32),
        pltpu.VMEM((CH, L), _f32),
        pltpu.VMEM((F,), _f32),
        pltpu.SemaphoreType.DMA,
        pltpu.SemaphoreType.DMA,
        pltpu.SemaphoreType.DMA,
        pltpu.SemaphoreType.DMA,
        pltpu.SemaphoreType.DMA,
        pltpu.SemaphoreType.DMA,
    ],
)


def _k1_body(x_ref, wn_ref, ws_ref, bs_ref, pre_ref, self_ref):
    xv = x_ref[...]
    pre_ref[...] = jnp.dot(xv, wn_ref[...], preferred_element_type=_f32)
    self_ref[...] = (jnp.dot(xv, ws_ref[...], preferred_element_type=_f32)
                     + bs_ref[...])


_k1 = pl.pallas_call(
    _k1_body,
    out_shape=(jax.ShapeDtypeStruct((NN, F), _f32),
               jax.ShapeDtypeStruct((NN, F), _f32)),
)


def _mix_body(self_ref, agg_ref, deg_ref, wn_ref, ws_ref, bs_ref,
              pre_ref, self2_ref):
    deg = deg_ref[0, :NN, 0:1] + deg_ref[1, :NN, 0:1]
    agg = agg_ref[0, :NN, :] + agg_ref[1, :NN, :]
    h = jnp.maximum(self_ref[...] + agg / jnp.maximum(deg, 1.0), 0.0)
    pre_ref[...] = jnp.dot(h, wn_ref[...], preferred_element_type=_f32)
    self2_ref[...] = (jnp.dot(h, ws_ref[...], preferred_element_type=_f32)
                      + bs_ref[...])


_k2 = pl.pallas_call(
    _mix_body,
    out_shape=(jax.ShapeDtypeStruct((NN, F), _f32),
               jax.ShapeDtypeStruct((NN, F), _f32)),
)


def _k3_body(self_ref, agg_ref, deg_ref, wa_ref, wb_ref, bm_ref,
             h_ref, a_ref, b_ref):
    deg = deg_ref[0, :NN, 0:1] + deg_ref[1, :NN, 0:1]
    agg = agg_ref[0, :NN, :] + agg_ref[1, :NN, :]
    h = jnp.maximum(self_ref[...] + agg / jnp.maximum(deg, 1.0), 0.0)
    h_ref[...] = h
    a_ref[...] = (jnp.dot(h, wa_ref[...], preferred_element_type=_f32)
                  + bm_ref[...])
    b_ref[...] = jnp.dot(h, wb_ref[...], preferred_element_type=_f32)


_k3 = pl.pallas_call(
    _k3_body,
    out_shape=(jax.ShapeDtypeStruct((NN, F), _f32),
               jax.ShapeDtypeStruct((NN, F), _f32),
               jax.ShapeDtypeStruct((NN, F), _f32)),
)


def _fin_body(p_ref, b2_ref, out_ref):
    i = lax.broadcasted_iota(jnp.int32, (128, 8), 0) // L
    j = lax.broadcasted_iota(jnp.int32, (128, 8), 1)
    sel = (i == j).astype(_f32)
    z = jnp.dot(p_ref[...], sel, preferred_element_type=_f32) + b2_ref[...]
    out_ref[...] = jax.nn.sigmoid(z)


_FINR = NE * L // 128
_fin = pl.pallas_call(
    _fin_body,
    grid=(5,),
    in_specs=[pl.BlockSpec((_FINR // 5, 128), lambda i: (i, 0)),
              pl.BlockSpec((1, 1), lambda i: (0, 0))],
    out_specs=pl.BlockSpec((_FINR // 5, 8), lambda i: (i, 0)),
    out_shape=jax.ShapeDtypeStruct((_FINR, 8), _f32),
)


def kernel(x, W_self1, b_self1, W_neigh1, W_self2, b_self2, W_neigh2,
           W_mlp1, b_mlp1, W_mlp2, b_mlp2, edge_index, score_edge_index):
    src = edge_index[0].astype(jnp.int32).reshape(NW, NCH, CH)
    dst = edge_index[1].astype(jnp.int32).reshape(NW, NCH, CH)
    ssrc = score_edge_index[0].astype(jnp.int32).reshape(NW, NCH, CH)
    sdst = score_edge_index[1].astype(jnp.int32).reshape(NW, NCH, CH)
    z64 = jnp.zeros((NPAD, F), _f32)
    z16 = jnp.zeros((NPAD, L), _f32)
    ones = jnp.ones((CH, L), _f32)

    pre1, self1 = _k1(x, W_neigh1, W_self1, b_self1.reshape(1, F))
    agg1, deg = _seg_deg(src, dst, pre1, z64, z16, ones)
    pre2, self2 = _k2(self1, agg1, deg, W_neigh2, W_self2,
                      b_self2.reshape(1, F))
    agg2 = _seg(src, dst, pre2, z64)
    h, A, B = _k3(self2, agg2, deg, W_mlp1[:F], W_mlp1[F:],
                  b_mlp1.reshape(1, F))
    parts = _score(ssrc, sdst, A, B, W_mlp2.reshape(F))
    score = _fin(parts.reshape(_FINR, 128), b_mlp2.reshape(1, 1))
    return (score.reshape(NE, 1), h)


# speedup vs baseline: 2.1612x; 1.0055x over previous
import functools

import jax
import jax.numpy as jnp
from jax import lax
from jax.experimental import pallas as pl
from jax.experimental.pallas import tpu as pltpu
from jax.experimental.pallas import tpu_sc as plsc

NN = 10000
NE = 320000
FIN = 128
F = 64
L = 16
NC, NS = 2, 16
NW = NC * NS
NPAD = 10240
RPT = NPAD // NS
EPW = NE // NW
CH = 80
NCH = EPW // CH
RPC = CH * L // 128

_f32 = jnp.float32
_mesh = plsc.VectorSubcoreMesh(core_axis_name="c", subcore_axis_name="s")
_sc_params = pltpu.CompilerParams(use_tc_tiling_on_sc=False)


def _seg_body(with_deg, src_hbm, dst_hbm, pre_hbm, z64_hbm, *rest):
    if with_deg:
        (z16_hbm, ones_hbm, agg_out, deg_out,
         idxs, idxd, r0b, r1b, r2b, ones_v, acc_sh, deg_sh,
         g0, g1, g2, s0, s1, s2, d0, d1, d2) = rest
        dsem = (d0, d1, d2)
    else:
        (agg_out, idxs, idxd, r0b, r1b, r2b, acc_sh,
         g0, g1, g2, s0, s1, s2) = rest
    rows = (r0b, r1b, r2b)
    gs = (g0, g1, g2)
    ss = (s0, s1, s2)
    c = lax.axis_index("c")
    s = lax.axis_index("s")
    wid = s * NC + c
    r0 = s * RPT
    pltpu.sync_copy(z64_hbm.at[pl.ds(r0, RPT)], acc_sh.at[pl.ds(r0, RPT)])
    if with_deg:
        pltpu.sync_copy(z16_hbm.at[pl.ds(r0, RPT)], deg_sh.at[pl.ds(r0, RPT)])
        pltpu.sync_copy(ones_hbm, ones_v)
    pltpu.sync_copy(src_hbm.at[wid], idxs)
    pltpu.sync_copy(dst_hbm.at[wid], idxd)
    plsc.subcore_barrier()

    def g_start(j, b):
        pltpu.async_copy(pre_hbm.at[idxs.at[j]], rows[b], gs[b])

    def g_wait(j, b):
        pltpu.make_async_copy(pre_hbm.at[idxs.at[j]], rows[b], gs[b]).wait()

    def d_wait(j, b):
        pltpu.make_async_copy(ones_v, deg_sh.at[idxd.at[j]], dsem[b]).wait()

    def s_start(j, b):
        pltpu.async_copy(rows[b], acc_sh.at[idxd.at[j]], ss[b], add=True)
        if with_deg:
            pltpu.async_copy(ones_v, deg_sh.at[idxd.at[j]], dsem[b],
                             add=True)

    def s_wait(j, b):
        pltpu.make_async_copy(rows[b], acc_sh.at[idxd.at[j]], ss[b]).wait()

    g_start(0, 0)
    g_start(1, 1)

    @pl.loop(0, NCH - 2, step=3)
    def _loop(j2):
        for b in range(3):
            j = j2 + b
            g_wait(j, b)
            if with_deg:
                @pl.when(j2 >= 3)
                def _():
                    d_wait(j - 3, b)
            s_start(j, b)
            nb = (b + 2) % 3
            if b == 0:
                @pl.when(j2 >= 1)
                def _():
                    s_wait(j - 1, nb)
            else:
                s_wait(j - 1, nb)
            g_start(j + 2, nb)

    for j in (NCH - 2, NCH - 1):
        b = j % 3
        g_wait(j, b)
        if with_deg:
            d_wait(j - 3, b)
        s_start(j, b)
    for j in (NCH - 3, NCH - 2, NCH - 1):
        s_wait(j, j % 3)
        if with_deg:
            d_wait(j, j % 3)
    plsc.subcore_barrier()
    pltpu.sync_copy(acc_sh.at[pl.ds(r0, RPT)], agg_out.at[c].at[pl.ds(r0, RPT)])
    if with_deg:
        pltpu.sync_copy(deg_sh.at[pl.ds(r0, RPT)], deg_out.at[c].at[pl.ds(r0, RPT)])


_seg_deg = pl.kernel(
    functools.partial(_seg_body, True),
    out_type=(jax.ShapeDtypeStruct((NC, NPAD, F), _f32),
              jax.ShapeDtypeStruct((NC, NPAD, L), _f32)),
    mesh=_mesh,
    compiler_params=_sc_params,
    scratch_types=[
        pltpu.VMEM((NCH, CH), jnp.int32),
        pltpu.VMEM((NCH, CH), jnp.int32),
        pltpu.VMEM((CH, F), _f32),
        pltpu.VMEM((CH, F), _f32),
        pltpu.VMEM((CH, F), _f32),
        pltpu.VMEM((CH, L), _f32),
        pltpu.VMEM_SHARED((NPAD, F), _f32),
        pltpu.VMEM_SHARED((NPAD, L), _f32),
        pltpu.SemaphoreType.DMA,
        pltpu.SemaphoreType.DMA,
        pltpu.SemaphoreType.DMA,
        pltpu.SemaphoreType.DMA,
        pltpu.SemaphoreType.DMA,
        pltpu.SemaphoreType.DMA,
        pltpu.SemaphoreType.DMA,
        pltpu.SemaphoreType.DMA,
        pltpu.SemaphoreType.DMA,
    ],
)

_seg = pl.kernel(
    functools.partial(_seg_body, False),
    out_type=jax.ShapeDtypeStruct((NC, NPAD, F), _f32),
    mesh=_mesh,
    compiler_params=_sc_params,
    scratch_types=[
        pltpu.VMEM((NCH, CH), jnp.int32),
        pltpu.VMEM((NCH, CH), jnp.int32),
        pltpu.VMEM((CH, F), _f32),
        pltpu.VMEM((CH, F), _f32),
        pltpu.VMEM((CH, F), _f32),
        pltpu.VMEM_SHARED((NPAD, F), _f32),
        pltpu.SemaphoreType.DMA,
        pltpu.SemaphoreType.DMA,
        pltpu.SemaphoreType.DMA,
        pltpu.SemaphoreType.DMA,
        pltpu.SemaphoreType.DMA,
        pltpu.SemaphoreType.DMA,
    ],
)


def _score_body(srcs_hbm, dsts_hbm, a_hbm, b_hbm, w2_hbm, out_hbm,
                idxs, idxd, ra0, ra1, rb0, rb1, p0, p1, w2_v,
                ga0, ga1, gb0, gb1, os0, os1):
    rows_a = (ra0, ra1)
    rows_b = (rb0, rb1)
    part = (p0, p1)
    gsa = (ga0, ga1)
    gsb = (gb0, gb1)
    osem = (os0, os1)
    c = lax.axis_index("c")
    s = lax.axis_index("s")
    wid = s * NC + c
    rbase = wid * (EPW * L // 128)
    pltpu.sync_copy(srcs_hbm.at[wid], idxs)
    pltpu.sync_copy(dsts_hbm.at[wid], idxd)
    pltpu.sync_copy(w2_hbm, w2_v)
    w2s = tuple(w2_v[pl.ds(k * L, L)] for k in range(F // L))

    def g_start(j, b):
        pltpu.async_copy(a_hbm.at[idxs.at[j]], rows_a[b], gsa[b])
        pltpu.async_copy(b_hbm.at[idxd.at[j]], rows_b[b], gsb[b])

    def g_wait(j, b):
        pltpu.make_async_copy(a_hbm.at[idxs.at[j]], rows_a[b], gsa[b]).wait()
        pltpu.make_async_copy(b_hbm.at[idxd.at[j]], rows_b[b], gsb[b]).wait()

    def compute(b):
        U = 8

        def edges(i, carry):
            accs = []
            for u in range(U):
                e = i * U + u
                acc = jnp.zeros((L,), _f32)
                for k in range(F // L):
                    g = jnp.maximum(
                        rows_a[b][e, pl.ds(k * L, L)]
                        + rows_b[b][e, pl.ds(k * L, L)], 0.0)
                    acc = acc + g * w2s[k]
                accs.append(acc)
            for u in range(U):
                part[b][i, pl.ds(u * L, L)] = accs[u]
            return carry

        lax.fori_loop(0, CH // U, edges, 0)

    def o_start(j, b):
        pltpu.async_copy(part[b], out_hbm.at[pl.ds(rbase + j * RPC, RPC)],
                         osem[b])

    def o_wait(j, b):
        pltpu.make_async_copy(part[b], out_hbm.at[pl.ds(rbase + j * RPC, RPC)],
                              osem[b]).wait()

    g_start(0, 0)
    g_start(1, 1)

    @pl.loop(0, NCH - 1, step=2)
    def _loop(j2):
        for b in range(2):
            j = j2 + b

            @pl.when(j2 >= 2)
            def _():
                o_wait(j - 2, b)

            g_wait(j, b)
            compute(b)
            o_start(j, b)
            if b == 0:
                g_start(j + 2, b)
            else:
                @pl.when(j2 + 3 < NCH)
                def _():
                    g_start(j + 2, b)

    j = NCH - 1
    o_wait(j - 2, 0)
    g_wait(j, 0)
    compute(0)
    o_start(j, 0)
    o_wait(j - 1, 1)
    o_wait(j, 0)


_score = pl.kernel(
    _score_body,
    out_type=jax.ShapeDtypeStruct((NE * L // 128, 128), _f32),
    mesh=_mesh,
    compiler_params=_sc_params,
    scratch_types=[
        pltpu.VMEM((NCH, CH), jnp.int32),
        pltpu.VMEM((NCH, CH), jnp.int32),
        pltpu.VMEM((CH, F), _f32),
        pltpu.VMEM((CH, F), _f32),
        pltpu.VMEM((CH, F), _f32),
        pltpu.VMEM((CH, F), _f32),
        pltpu.VMEM((RPC, 128), _f32),
        pltpu.VMEM((RPC, 128), _f32),
        pltpu.VMEM((F,), _f32),
        pltpu.SemaphoreType.DMA,
        pltpu.SemaphoreType.DMA,
        pltpu.SemaphoreType.DMA,
        pltpu.SemaphoreType.DMA,
        pltpu.SemaphoreType.DMA,
        pltpu.SemaphoreType.DMA,
    ],
)


def _k1_body(x_ref, wn_ref, ws_ref, bs_ref, pre_ref, self_ref):
    xv = x_ref[...]
    pre_ref[...] = jnp.dot(xv, wn_ref[...], preferred_element_type=_f32)
    self_ref[...] = (jnp.dot(xv, ws_ref[...], preferred_element_type=_f32)
                     + bs_ref[...])


_k1 = pl.pallas_call(
    _k1_body,
    out_shape=(jax.ShapeDtypeStruct((NN, F), _f32),
               jax.ShapeDtypeStruct((NN, F), _f32)),
)


def _mix_body(self_ref, agg_ref, deg_ref, wn_ref, ws_ref, bs_ref,
              pre_ref, self2_ref):
    deg = deg_ref[0, :NN, 0:1] + deg_ref[1, :NN, 0:1]
    agg = agg_ref[0, :NN, :] + agg_ref[1, :NN, :]
    h = jnp.maximum(self_ref[...] + agg / jnp.maximum(deg, 1.0), 0.0)
    pre_ref[...] = jnp.dot(h, wn_ref[...], preferred_element_type=_f32)
    self2_ref[...] = (jnp.dot(h, ws_ref[...], preferred_element_type=_f32)
                      + bs_ref[...])


_k2 = pl.pallas_call(
    _mix_body,
    out_shape=(jax.ShapeDtypeStruct((NN, F), _f32),
               jax.ShapeDtypeStruct((NN, F), _f32)),
)


def _k3_body(self_ref, agg_ref, deg_ref, wa_ref, wb_ref, bm_ref,
             h_ref, a_ref, b_ref):
    deg = deg_ref[0, :NN, 0:1] + deg_ref[1, :NN, 0:1]
    agg = agg_ref[0, :NN, :] + agg_ref[1, :NN, :]
    h = jnp.maximum(self_ref[...] + agg / jnp.maximum(deg, 1.0), 0.0)
    h_ref[...] = h
    a_ref[...] = (jnp.dot(h, wa_ref[...], preferred_element_type=_f32)
                  + bm_ref[...])
    b_ref[...] = jnp.dot(h, wb_ref[...], preferred_element_type=_f32)


_k3 = pl.pallas_call(
    _k3_body,
    out_shape=(jax.ShapeDtypeStruct((NN, F), _f32),
               jax.ShapeDtypeStruct((NN, F), _f32),
               jax.ShapeDtypeStruct((NN, F), _f32)),
)


def _fin_body(p_ref, b2_ref, out_ref):
    i = lax.broadcasted_iota(jnp.int32, (128, 8), 0) // L
    j = lax.broadcasted_iota(jnp.int32, (128, 8), 1)
    sel = (i == j).astype(_f32)
    z = jnp.dot(p_ref[...], sel, preferred_element_type=_f32) + b2_ref[...]
    out_ref[...] = jax.nn.sigmoid(z)


_FINR = NE * L // 128
_fin = pl.pallas_call(
    _fin_body,
    grid=(5,),
    in_specs=[pl.BlockSpec((_FINR // 5, 128), lambda i: (i, 0)),
              pl.BlockSpec((1, 1), lambda i: (0, 0))],
    out_specs=pl.BlockSpec((_FINR // 5, 8), lambda i: (i, 0)),
    out_shape=jax.ShapeDtypeStruct((_FINR, 8), _f32),
)


def kernel(x, W_self1, b_self1, W_neigh1, W_self2, b_self2, W_neigh2,
           W_mlp1, b_mlp1, W_mlp2, b_mlp2, edge_index, score_edge_index):
    src = edge_index[0].astype(jnp.int32).reshape(NW, NCH, CH)
    dst = edge_index[1].astype(jnp.int32).reshape(NW, NCH, CH)
    ssrc = score_edge_index[0].astype(jnp.int32).reshape(NW, NCH, CH)
    sdst = score_edge_index[1].astype(jnp.int32).reshape(NW, NCH, CH)
    z64 = jnp.zeros((NPAD, F), _f32)
    z16 = jnp.zeros((NPAD, L), _f32)
    ones = jnp.ones((CH, L), _f32)

    pre1, self1 = _k1(x, W_neigh1, W_self1, b_self1.reshape(1, F))
    agg1, deg = _seg_deg(src, dst, pre1, z64, z16, ones)
    pre2, self2 = _k2(self1, agg1, deg, W_neigh2, W_self2,
                      b_self2.reshape(1, F))
    agg2 = _seg(src, dst, pre2, z64)
    h, A, B = _k3(self2, agg2, deg, W_mlp1[:F], W_mlp1[F:],
                  b_mlp1.reshape(1, F))
    parts = _score(ssrc, sdst, A, B, W_mlp2.reshape(F))
    score = _fin(parts, b_mlp2.reshape(1, 1))
    return (score.reshape(NE, 1), h)


# speedup vs baseline: 2.1615x; 1.0001x over previous
import functools

import jax
import jax.numpy as jnp
from jax import lax
from jax.experimental import pallas as pl
from jax.experimental.pallas import tpu as pltpu
from jax.experimental.pallas import tpu_sc as plsc

NN = 10000
NE = 320000
FIN = 128
F = 64
L = 16
NC, NS = 2, 16
NW = NC * NS
NPAD = 10240
RPT = NPAD // NS
EPW = NE // NW
CH = 80
NCH = EPW // CH
RPC = CH * L // 128

_f32 = jnp.float32
_mesh = plsc.VectorSubcoreMesh(core_axis_name="c", subcore_axis_name="s")
_sc_params = pltpu.CompilerParams(use_tc_tiling_on_sc=False)


def _seg_body(with_deg, src_hbm, dst_hbm, pre_hbm, z64_hbm, *rest):
    if with_deg:
        (z16_hbm, ones_hbm, agg_out, deg_out,
         idxs, idxd, r0b, r1b, r2b, ones_v, acc_sh, deg_sh,
         g0, g1, g2, s0, s1, s2, d0, d1, d2) = rest
        dsem = (d0, d1, d2)
    else:
        (agg_out, idxs, idxd, r0b, r1b, r2b, acc_sh,
         g0, g1, g2, s0, s1, s2) = rest
    rows = (r0b, r1b, r2b)
    gs = (g0, g1, g2)
    ss = (s0, s1, s2)
    c = lax.axis_index("c")
    s = lax.axis_index("s")
    wid = s * NC + c
    r0 = s * RPT
    pltpu.sync_copy(z64_hbm.at[pl.ds(r0, RPT)], acc_sh.at[pl.ds(r0, RPT)])
    if with_deg:
        pltpu.sync_copy(z16_hbm.at[pl.ds(r0, RPT)], deg_sh.at[pl.ds(r0, RPT)])
        pltpu.sync_copy(ones_hbm, ones_v)
    pltpu.sync_copy(src_hbm.at[wid], idxs)
    pltpu.sync_copy(dst_hbm.at[wid], idxd)
    plsc.subcore_barrier()

    def g_start(j, b):
        pltpu.async_copy(pre_hbm.at[idxs.at[j]], rows[b], gs[b])

    def g_wait(j, b):
        pltpu.make_async_copy(pre_hbm.at[idxs.at[j]], rows[b], gs[b]).wait()

    def d_wait(j, b):
        pltpu.make_async_copy(ones_v, deg_sh.at[idxd.at[j]], dsem[b]).wait()

    def s_start(j, b):
        pltpu.async_copy(rows[b], acc_sh.at[idxd.at[j]], ss[b], add=True)
        if with_deg:
            pltpu.async_copy(ones_v, deg_sh.at[idxd.at[j]], dsem[b],
                             add=True)

    def s_wait(j, b):
        pltpu.make_async_copy(rows[b], acc_sh.at[idxd.at[j]], ss[b]).wait()

    g_start(0, 0)
    g_start(1, 1)

    @pl.loop(0, NCH - 2, step=3)
    def _loop(j2):
        for b in range(3):
            j = j2 + b
            g_wait(j, b)
            if with_deg:
                @pl.when(j2 >= 3)
                def _():
                    d_wait(j - 3, b)
            s_start(j, b)
            nb = (b + 2) % 3
            if b == 0:
                @pl.when(j2 >= 1)
                def _():
                    s_wait(j - 1, nb)
            else:
                s_wait(j - 1, nb)
            g_start(j + 2, nb)

    for j in (NCH - 2, NCH - 1):
        b = j % 3
        g_wait(j, b)
        if with_deg:
            d_wait(j - 3, b)
        s_start(j, b)
    for j in (NCH - 3, NCH - 2, NCH - 1):
        s_wait(j, j % 3)
        if with_deg:
            d_wait(j, j % 3)
    plsc.subcore_barrier()
    pltpu.sync_copy(acc_sh.at[pl.ds(r0, RPT)], agg_out.at[c].at[pl.ds(r0, RPT)])
    if with_deg:
        pltpu.sync_copy(deg_sh.at[pl.ds(r0, RPT)], deg_out.at[c].at[pl.ds(r0, RPT)])


_seg_deg = pl.kernel(
    functools.partial(_seg_body, True),
    out_type=(jax.ShapeDtypeStruct((NC, NPAD, F), _f32),
              jax.ShapeDtypeStruct((NC, NPAD, L), _f32)),
    mesh=_mesh,
    compiler_params=_sc_params,
    scratch_types=[
        pltpu.VMEM((NCH, CH), jnp.int32),
        pltpu.VMEM((NCH, CH), jnp.int32),
        pltpu.VMEM((CH, F), _f32),
        pltpu.VMEM((CH, F), _f32),
        pltpu.VMEM((CH, F), _f32),
        pltpu.VMEM((CH, L), _f32),
        pltpu.VMEM_SHARED((NPAD, F), _f32),
        pltpu.VMEM_SHARED((NPAD, L), _f32),
        pltpu.SemaphoreType.DMA,
        pltpu.SemaphoreType.DMA,
        pltpu.SemaphoreType.DMA,
        pltpu.SemaphoreType.DMA,
        pltpu.SemaphoreType.DMA,
        pltpu.SemaphoreType.DMA,
        pltpu.SemaphoreType.DMA,
        pltpu.SemaphoreType.DMA,
        pltpu.SemaphoreType.DMA,
    ],
)

_seg = pl.kernel(
    functools.partial(_seg_body, False),
    out_type=jax.ShapeDtypeStruct((NC, NPAD, F), _f32),
    mesh=_mesh,
    compiler_params=_sc_params,
    scratch_types=[
        pltpu.VMEM((NCH, CH), jnp.int32),
        pltpu.VMEM((NCH, CH), jnp.int32),
        pltpu.VMEM((CH, F), _f32),
        pltpu.VMEM((CH, F), _f32),
        pltpu.VMEM((CH, F), _f32),
        pltpu.VMEM_SHARED((NPAD, F), _f32),
        pltpu.SemaphoreType.DMA,
        pltpu.SemaphoreType.DMA,
        pltpu.SemaphoreType.DMA,
        pltpu.SemaphoreType.DMA,
        pltpu.SemaphoreType.DMA,
        pltpu.SemaphoreType.DMA,
    ],
)


def _score_body(srcs_hbm, dsts_hbm, a_hbm, b_hbm, w2_hbm, out_hbm,
                idxs, idxd, ra0, ra1, rb0, rb1, p0, p1, w2_v,
                ga0, ga1, gb0, gb1, os0, os1):
    rows_a = (ra0, ra1)
    rows_b = (rb0, rb1)
    part = (p0, p1)
    gsa = (ga0, ga1)
    gsb = (gb0, gb1)
    osem = (os0, os1)
    c = lax.axis_index("c")
    s = lax.axis_index("s")
    wid = s * NC + c
    rbase = wid * (EPW * L // 128)
    pltpu.sync_copy(srcs_hbm.at[wid], idxs)
    pltpu.sync_copy(dsts_hbm.at[wid], idxd)
    pltpu.sync_copy(w2_hbm, w2_v)
    w2s = tuple(w2_v[pl.ds(k * L, L)] for k in range(F // L))

    def g_start(j, b):
        pltpu.async_copy(a_hbm.at[idxs.at[j]], rows_a[b], gsa[b])
        pltpu.async_copy(b_hbm.at[idxd.at[j]], rows_b[b], gsb[b])

    def g_wait(j, b):
        pltpu.make_async_copy(a_hbm.at[idxs.at[j]], rows_a[b], gsa[b]).wait()
        pltpu.make_async_copy(b_hbm.at[idxd.at[j]], rows_b[b], gsb[b]).wait()

    def compute(b):
        U = 16

        def edges(i, carry):
            accs = []
            for u in range(U):
                e = i * U + u
                acc = jnp.zeros((L,), _f32)
                for k in range(F // L):
                    g = jnp.maximum(
                        rows_a[b][e, pl.ds(k * L, L)]
                        + rows_b[b][e, pl.ds(k * L, L)], 0.0)
                    acc = acc + g * w2s[k]
                accs.append(acc)
            for u in range(U):
                part[b][i * (U // 8) + u // 8, pl.ds((u % 8) * L, L)] = accs[u]
            return carry

        lax.fori_loop(0, CH // U, edges, 0)

    def o_start(j, b):
        pltpu.async_copy(part[b], out_hbm.at[pl.ds(rbase + j * RPC, RPC)],
                         osem[b])

    def o_wait(j, b):
        pltpu.make_async_copy(part[b], out_hbm.at[pl.ds(rbase + j * RPC, RPC)],
                              osem[b]).wait()

    g_start(0, 0)
    g_start(1, 1)

    @pl.loop(0, NCH - 1, step=2)
    def _loop(j2):
        for b in range(2):
            j = j2 + b

            @pl.when(j2 >= 2)
            def _():
                o_wait(j - 2, b)

            g_wait(j, b)
            compute(b)
            o_start(j, b)
            if b == 0:
                g_start(j + 2, b)
            else:
                @pl.when(j2 + 3 < NCH)
                def _():
                    g_start(j + 2, b)

    j = NCH - 1
    o_wait(j - 2, 0)
    g_wait(j, 0)
    compute(0)
    o_start(j, 0)
    o_wait(j - 1, 1)
    o_wait(j, 0)


_score = pl.kernel(
    _score_body,
    out_type=jax.ShapeDtypeStruct((NE * L // 128, 128), _f32),
    mesh=_mesh,
    compiler_params=_sc_params,
    scratch_types=[
        pltpu.VMEM((NCH, CH), jnp.int32),
        pltpu.VMEM((NCH, CH), jnp.int32),
        pltpu.VMEM((CH, F), _f32),
        pltpu.VMEM((CH, F), _f32),
        pltpu.VMEM((CH, F), _f32),
        pltpu.VMEM((CH, F), _f32),
        pltpu.VMEM((RPC, 128), _f32),
        pltpu.VMEM((RPC, 128), _f32),
        pltpu.VMEM((F,), _f32),
        pltpu.SemaphoreType.DMA,
        pltpu.SemaphoreType.DMA,
        pltpu.SemaphoreType.DMA,
        pltpu.SemaphoreType.DMA,
        pltpu.SemaphoreType.DMA,
        pltpu.SemaphoreType.DMA,
    ],
)


def _k1_body(x_ref, wn_ref, ws_ref, bs_ref, pre_ref, self_ref):
    xv = x_ref[...]
    pre_ref[...] = jnp.dot(xv, wn_ref[...], preferred_element_type=_f32)
    self_ref[...] = (jnp.dot(xv, ws_ref[...], preferred_element_type=_f32)
                     + bs_ref[...])


_k1 = pl.pallas_call(
    _k1_body,
    out_shape=(jax.ShapeDtypeStruct((NN, F), _f32),
               jax.ShapeDtypeStruct((NN, F), _f32)),
)


def _mix_body(self_ref, agg_ref, deg_ref, wn_ref, ws_ref, bs_ref,
              pre_ref, self2_ref):
    deg = deg_ref[0, :NN, 0:1] + deg_ref[1, :NN, 0:1]
    agg = agg_ref[0, :NN, :] + agg_ref[1, :NN, :]
    h = jnp.maximum(self_ref[...] + agg / jnp.maximum(deg, 1.0), 0.0)
    pre_ref[...] = jnp.dot(h, wn_ref[...], preferred_element_type=_f32)
    self2_ref[...] = (jnp.dot(h, ws_ref[...], preferred_element_type=_f32)
                      + bs_ref[...])


_k2 = pl.pallas_call(
    _mix_body,
    out_shape=(jax.ShapeDtypeStruct((NN, F), _f32),
               jax.ShapeDtypeStruct((NN, F), _f32)),
)


def _k3_body(self_ref, agg_ref, deg_ref, wa_ref, wb_ref, bm_ref,
             h_ref, a_ref, b_ref):
    deg = deg_ref[0, :NN, 0:1] + deg_ref[1, :NN, 0:1]
    agg = agg_ref[0, :NN, :] + agg_ref[1, :NN, :]
    h = jnp.maximum(self_ref[...] + agg / jnp.maximum(deg, 1.0), 0.0)
    h_ref[...] = h
    a_ref[...] = (jnp.dot(h, wa_ref[...], preferred_element_type=_f32)
                  + bm_ref[...])
    b_ref[...] = jnp.dot(h, wb_ref[...], preferred_element_type=_f32)


_k3 = pl.pallas_call(
    _k3_body,
    out_shape=(jax.ShapeDtypeStruct((NN, F), _f32),
               jax.ShapeDtypeStruct((NN, F), _f32),
               jax.ShapeDtypeStruct((NN, F), _f32)),
)


def _fin_body(p_ref, b2_ref, out_ref):
    i = lax.broadcasted_iota(jnp.int32, (128, 8), 0) // L
    j = lax.broadcasted_iota(jnp.int32, (128, 8), 1)
    sel = (i == j).astype(_f32)
    z = jnp.dot(p_ref[...], sel, preferred_element_type=_f32) + b2_ref[...]
    out_ref[...] = jax.nn.sigmoid(z)


_FINR = NE * L // 128
_fin = pl.pallas_call(
    _fin_body,
    grid=(5,),
    in_specs=[pl.BlockSpec((_FINR // 5, 128), lambda i: (i, 0)),
              pl.BlockSpec((1, 1), lambda i: (0, 0))],
    out_specs=pl.BlockSpec((_FINR // 5, 8), lambda i: (i, 0)),
    out_shape=jax.ShapeDtypeStruct((_FINR, 8), _f32),
)


def kernel(x, W_self1, b_self1, W_neigh1, W_self2, b_self2, W_neigh2,
           W_mlp1, b_mlp1, W_mlp2, b_mlp2, edge_index, score_edge_index):
    src = edge_index[0].astype(jnp.int32).reshape(NW, NCH, CH)
    dst = edge_index[1].astype(jnp.int32).reshape(NW, NCH, CH)
    ssrc = score_edge_index[0].astype(jnp.int32).reshape(NW, NCH, CH)
    sdst = score_edge_index[1].astype(jnp.int32).reshape(NW, NCH, CH)
    z64 = jnp.zeros((NPAD, F), _f32)
    z16 = jnp.zeros((NPAD, L), _f32)
    ones = jnp.ones((CH, L), _f32)

    pre1, self1 = _k1(x, W_neigh1, W_self1, b_self1.reshape(1, F))
    agg1, deg = _seg_deg(src, dst, pre1, z64, z16, ones)
    pre2, self2 = _k2(self1, agg1, deg, W_neigh2, W_self2,
                      b_self2.reshape(1, F))
    agg2 = _seg(src, dst, pre2, z64)
    h, A, B = _k3(self2, agg2, deg, W_mlp1[:F], W_mlp1[F:],
                  b_mlp1.reshape(1, F))
    parts = _score(ssrc, sdst, A, B, W_mlp2.reshape(F))
    score = _fin(parts, b_mlp2.reshape(1, 1))
    return (score.reshape(NE, 1), h)
